# baseline (speedup 1.0000x reference)
import jax
import jax.numpy as jnp
from jax.experimental import pallas as pl

NV = 32
CH = 8
GPR = 32
R = 1024


def _onehot_body(x_ref, o_ref):
    X = x_ref[...]
    i = jax.lax.broadcasted_iota(jnp.int32, (GPR * CH, NV), 0)
    k = jax.lax.broadcasted_iota(jnp.int32, (GPR * CH, NV), 1)
    Q = (i == CH * k + (CH - 1)).astype(jnp.float32)
    idx = jax.lax.dot_general(X, Q, (((1,), (0,)), ((), ())),
                              preferred_element_type=jnp.float32)
    k2 = jax.lax.broadcasted_iota(jnp.int32, (NV, GPR * NV), 0)
    j2 = jax.lax.broadcasted_iota(jnp.int32, (NV, GPR * NV), 1)
    Bm = ((j2 >> 5) == k2).astype(jnp.float32)
    E = jax.lax.dot_general(idx, Bm, (((1,), (0,)), ((), ())),
                            preferred_element_type=jnp.float32)
    v = (jax.lax.broadcasted_iota(jnp.int32, (R, GPR * NV), 1) & (NV - 1)
         ).astype(jnp.float32)
    o_ref[...] = jnp.where(E == v, 1.0, 0.0)


def kernel(network):
    B, T, C = network.shape
    N = B * T
    rows = N // GPR
    x = network.reshape(rows, GPR * C)
    out = pl.pallas_call(
        _onehot_body,
        grid=(rows // R,),
        in_specs=[pl.BlockSpec((R, GPR * C), lambda i: (i, 0))],
        out_specs=pl.BlockSpec((R, GPR * NV), lambda i: (i, 0)),
        out_shape=jax.ShapeDtypeStruct((rows, GPR * NV), jnp.float32),
    )(x)
    return out.reshape(B, T, NV)


# speedup vs baseline: 1.0265x; 1.0265x over previous
import jax
import jax.numpy as jnp
from jax.experimental import pallas as pl

NV = 32
CH = 8
RI = 2048


def _onehot_body(x_ref, o_ref):
    X = x_ref[...]
    i4 = jax.lax.broadcasted_iota(jnp.int32, (128, 64), 0)
    c4 = jax.lax.broadcasted_iota(jnp.int32, (128, 64), 1)
    Q4 = (i4 == CH * (c4 & 15) + (CH - 1)).astype(jnp.float32)
    T = jax.lax.dot_general(X, Q4, (((1,), (0,)), ((), ())),
                            preferred_element_type=jnp.float32)
    rep = jnp.repeat(T, 4, axis=0)
    rowp = jax.lax.broadcasted_iota(jnp.int32, (4 * RI, 64), 0) & 3
    lanep = jax.lax.broadcasted_iota(jnp.int32, (4 * RI, 64), 1) >> 4
    idxc4 = jnp.where(rowp == lanep, rep, 0.0)
    r2 = jax.lax.broadcasted_iota(jnp.int32, (64, 128), 0)
    j2 = jax.lax.broadcasted_iota(jnp.int32, (64, 128), 1)
    B2 = ((r2 & 15) == 4 * (r2 >> 4) + (j2 >> 5)).astype(jnp.float32)
    E = jax.lax.dot_general(idxc4, B2, (((1,), (0,)), ((), ())),
                            preferred_element_type=jnp.float32)
    v = (jax.lax.broadcasted_iota(jnp.int32, (4 * RI, 128), 1) & (NV - 1)
         ).astype(jnp.float32)
    o_ref[...] = jnp.where(E == v, 1.0, 0.0)


def kernel(network):
    B, T, C = network.shape
    N = B * T
    rows_in = N * C // 128
    rows_out = N * NV // 128
    x = network.reshape(rows_in, 128)
    out = pl.pallas_call(
        _onehot_body,
        grid=(rows_in // RI,),
        in_specs=[pl.BlockSpec((RI, 128), lambda i: (i, 0))],
        out_specs=pl.BlockSpec((4 * RI, 128), lambda i: (i, 0)),
        out_shape=jax.ShapeDtypeStruct((rows_out, 128), jnp.float32),
    )(x)
    return out.reshape(B, T, NV)


# speedup vs baseline: 19.6536x; 19.1454x over previous
import jax
import jax.numpy as jnp
from jax.experimental import pallas as pl

NV = 32
CH = 8
TB = 4


def _onehot_body(x_ref, o_ref):
    idx = x_ref[:, CH - 1:CH, :].astype(jnp.int32)
    v = jax.lax.broadcasted_iota(jnp.int32, o_ref.shape, 1)
    o_ref[...] = jnp.where(v == idx, 1.0, 0.0)


def kernel(network):
    B, T, C = network.shape
    xp = jnp.transpose(network, (1, 2, 0))
    out = pl.pallas_call(
        _onehot_body,
        grid=(T // TB,),
        in_specs=[pl.BlockSpec((TB, C, B), lambda i: (i, 0, 0))],
        out_specs=pl.BlockSpec((TB, NV, B), lambda i: (i, 0, 0)),
        out_shape=jax.ShapeDtypeStruct((T, NV, B), jnp.float32),
    )(xp)
    return jnp.transpose(out, (2, 0, 1))


# speedup vs baseline: 20.1498x; 1.0253x over previous
import jax
import jax.numpy as jnp
from jax.experimental import pallas as pl

NV = 32
CH = 8
TB = 8


def _onehot_body(x_ref, o_ref):
    idx = x_ref[:, CH - 1:CH, :].astype(jnp.int32)
    v = jax.lax.broadcasted_iota(jnp.int32, o_ref.shape, 1)
    o_ref[...] = jnp.where(v == idx, 1.0, 0.0)


def kernel(network):
    B, T, C = network.shape
    xp = jnp.transpose(network, (1, 2, 0))
    out = pl.pallas_call(
        _onehot_body,
        grid=(T // TB,),
        in_specs=[pl.BlockSpec((TB, C, B), lambda i: (i, 0, 0))],
        out_specs=pl.BlockSpec((TB, NV, B), lambda i: (i, 0, 0)),
        out_shape=jax.ShapeDtypeStruct((T, NV, B), jnp.float32),
    )(xp)
    return jnp.transpose(out, (2, 0, 1))


# speedup vs baseline: 20.3556x; 1.0102x over previous
import jax
import jax.numpy as jnp
from jax.experimental import pallas as pl

NV = 32
CH = 8
TB = 10


def _onehot_body(x_ref, o_ref):
    idx = x_ref[:, CH - 1:CH, :].astype(jnp.int32)
    v = jax.lax.broadcasted_iota(jnp.int32, o_ref.shape, 1)
    o_ref[...] = jnp.where(v == idx, 1.0, 0.0)


def kernel(network):
    B, T, C = network.shape
    xp = jnp.transpose(network, (1, 2, 0))
    out = pl.pallas_call(
        _onehot_body,
        grid=(T // TB,),
        in_specs=[pl.BlockSpec((TB, C, B), lambda i: (i, 0, 0))],
        out_specs=pl.BlockSpec((TB, NV, B), lambda i: (i, 0, 0)),
        out_shape=jax.ShapeDtypeStruct((T, NV, B), jnp.float32),
    )(xp)
    return jnp.transpose(out, (2, 0, 1))


# speedup vs baseline: 24.6181x; 1.2094x over previous
import jax
import jax.numpy as jnp
from jax.experimental import pallas as pl
from jax.experimental.pallas import tpu as pltpu

NV = 32
CH = 8
TB = 10


def _in_copy(x_hbm, scr, sems, step, slot):
    return pltpu.make_async_copy(
        x_hbm.at[pl.ds(step * TB, TB), CH - 1:CH, :],
        scr.at[slot],
        sems.at[slot],
    )


def _onehot_body(x_hbm, o_ref, scr, sems):
    i = pl.program_id(0)
    slot = i % 2

    @pl.when(i == 0)
    def _():
        _in_copy(x_hbm, scr, sems, i, slot).start()

    @pl.when(i + 1 < pl.num_programs(0))
    def _():
        _in_copy(x_hbm, scr, sems, i + 1, 1 - slot).start()

    _in_copy(x_hbm, scr, sems, i, slot).wait()
    idx = scr[slot].astype(jnp.int32)
    v = jax.lax.broadcasted_iota(jnp.int32, o_ref.shape, 1)
    o_ref[...] = jnp.where(v == idx, 1.0, 0.0)


def kernel(network):
    B, T, C = network.shape
    xp = jnp.transpose(network, (1, 2, 0))
    out = pl.pallas_call(
        _onehot_body,
        grid=(T // TB,),
        in_specs=[pl.BlockSpec(memory_space=pl.MemorySpace.ANY)],
        out_specs=pl.BlockSpec((TB, NV, B), lambda i: (i, 0, 0)),
        out_shape=jax.ShapeDtypeStruct((T, NV, B), jnp.float32),
        scratch_shapes=[
            pltpu.VMEM((2, TB, 1, B), jnp.float32),
            pltpu.SemaphoreType.DMA((2,)),
        ],
    )(xp)
    return jnp.transpose(out, (2, 0, 1))
